# baseline (speedup 1.0000x reference)
import functools

import jax
import jax.numpy as jnp
from jax import lax
from jax.experimental import pallas as pl
from jax.experimental.pallas import tpu as pltpu
from jax.experimental.pallas import tpu_sc as plsc

_N = 1024
_D = 128
_P = 4096
_R = 3
_K = 16
_RK = _R * _K
_CURRENT_TIME = 200.0

_NW = 32
_NODES_W = _N // _NW
_PAIRS_W = _P // _NW
_NSTEP = 4
_NPS = _NODES_W // _NSTEP
_ROWS = _NPS * _RK
_PAIR_BLK = 512


def _make_sc_kernel():
    mesh = plsc.VectorSubcoreMesh(core_axis_name="c", subcore_axis_name="s")

    @functools.partial(
        pl.kernel, mesh=mesh,
        compiler_params=pltpu.CompilerParams(
            needs_layout_passes=False, use_tc_tiling_on_sc=False),
        out_type=[jax.ShapeDtypeStruct((_R * _N, _D), jnp.float32),
                  jax.ShapeDtypeStruct((_P, _D), jnp.float32)],
        scratch_types=[
            pltpu.VMEM((_NODES_W * _RK,), jnp.int32),
            pltpu.VMEM((_ROWS, _D // 2), jnp.int32),
            pltpu.VMEM((_ROWS, _D // 2), jnp.int32),
            pltpu.VMEM((2 * _PAIRS_W,), jnp.int32),
            pltpu.VMEM((2 * _PAIRS_W, _D // 2), jnp.int32),
            pltpu.VMEM((_PAIRS_W, _D), jnp.float32),
            pltpu.VMEM((_R * _NODES_W, _D), jnp.float32),
            pltpu.SemaphoreType.DMA,
            pltpu.SemaphoreType.DMA,
            pltpu.SemaphoreType.DMA,
            pltpu.SemaphoreType.DMA,
        ],
    )
    def sc_body(ebf_hbm, idx_hbm, mn_hbm, g_hbm, de_hbm,
                idx_v, rows0, rows1, mn_v, prow_v, de_v, gbuf,
                sem0, sem1, semp0, semp1):
        wid = lax.axis_index("s") * 2 + lax.axis_index("c")
        pltpu.sync_copy(mn_hbm.at[pl.ds(wid * 2 * _PAIRS_W, 2 * _PAIRS_W)],
                        mn_v)
        cpp0 = pltpu.async_copy(ebf_hbm.at[mn_v.at[pl.ds(0, _PAIRS_W)]],
                                prow_v.at[pl.ds(0, _PAIRS_W)], semp0)
        cpp1 = pltpu.async_copy(
            ebf_hbm.at[mn_v.at[pl.ds(_PAIRS_W, _PAIRS_W)]],
            prow_v.at[pl.ds(_PAIRS_W, _PAIRS_W)], semp1)
        pltpu.sync_copy(idx_hbm.at[pl.ds(wid * _NODES_W * _RK,
                                         _NODES_W * _RK)], idx_v)

        bufs = (rows0, rows1)
        sems = (sem0, sem1)

        def fire(t):
            return pltpu.async_copy(
                ebf_hbm.at[idx_v.at[pl.ds(t * _ROWS, _ROWS)]],
                bufs[t % 2], sems[t % 2])

        cps = {0: fire(0), 1: fire(1)}
        for t in range(_NSTEP):
            cps[t].wait()
            buf = bufs[t % 2]

            def grp_body(g, _, _buf=buf, _t=t):
                u = g // _R
                r = g - u * _R
                base = g * _K

                def row_blk(j, accs, _buf=_buf, _base=base):
                    out = list(accs)
                    for jj in range(4):
                        row = _base + j * 4 + jj
                        for c in range(4):
                            w = _buf[row, pl.ds(c * 16, 16)]
                            v = plsc.bitcast(w, jnp.bfloat16)
                            a, b = plsc.unpack(
                                v, format=plsc.PackFormat.INTERLEAVED)
                            out[2 * c] = out[2 * c] + a
                            out[2 * c + 1] = out[2 * c + 1] + b
                    return tuple(out)

                accs = tuple(jnp.zeros((16,), jnp.float32) for _ in range(8))
                accs = lax.fori_loop(0, _K // 4, row_blk, accs)
                orow = r * _NODES_W + _t * _NPS + u
                for c in range(8):
                    gbuf[orow, pl.ds(c * 16, 16)] = accs[c]
                return 0

            lax.fori_loop(0, _NPS * _R, grp_body, 0)
            if t + 2 < _NSTEP:
                cps[t + 2] = fire(t + 2)

        for r in range(_R):
            pltpu.sync_copy(
                gbuf.at[pl.ds(r * _NODES_W, _NODES_W)],
                g_hbm.at[pl.ds(r * _N + wid * _NODES_W, _NODES_W)])

        cpp0.wait()
        cpp1.wait()

        def pair_body(p, _):
            for c in range(4):
                wa = prow_v[2 * p, pl.ds(c * 16, 16)]
                wb = prow_v[2 * p + 1, pl.ds(c * 16, 16)]
                a1, a2 = plsc.unpack(plsc.bitcast(wa, jnp.bfloat16),
                                     format=plsc.PackFormat.INTERLEAVED)
                b1, b2 = plsc.unpack(plsc.bitcast(wb, jnp.bfloat16),
                                     format=plsc.PackFormat.INTERLEAVED)
                de_v[p, pl.ds(c * 32, 16)] = a1 - b1
                de_v[p, pl.ds(c * 32 + 16, 16)] = a2 - b2
            return 0

        lax.fori_loop(0, _PAIRS_W, pair_body, 0)
        pltpu.sync_copy(de_v, de_hbm.at[pl.ds(wid * _PAIRS_W, _PAIRS_W)])

    return sc_body


_sc_stage = _make_sc_kernel()


def _dense_stage(ev_ref, g_ref, de_ref, pr_ref, Wp_ref, Wb_ref, bb_ref,
                 theta_ref, q1_ref, q2_ref, out_ref):
    theta = theta_ref[0, 0]
    alpha = jnp.sum(jnp.exp(-theta * (_CURRENT_TIME - ev_ref[...])))
    x = jnp.zeros((_N, _D), jnp.float32)
    for r in range(_R):
        x = x + jnp.dot(g_ref[r * _N:(r + 1) * _N, :], Wb_ref[r],
                        preferred_element_type=jnp.float32)
    bbar = jnp.mean(bb_ref[...], axis=0, keepdims=True)
    x = x * (1.0 / _RK) + bbar
    s = jnp.mean(jax.nn.sigmoid(x), axis=1, keepdims=True)
    diffs = jnp.dot(de_ref[...], Wp_ref[...],
                    preferred_element_type=jnp.float32)
    d2 = jnp.sum(diffs * diffs, axis=1, keepdims=True)
    q1 = q1_ref[0, 0]
    q2 = q2_ref[0, 0]
    for blk in range(_P // _PAIR_BLK):
        lo, hi = blk * _PAIR_BLK, (blk + 1) * _PAIR_BLK
        iota = lax.broadcasted_iota(jnp.int32, (_PAIR_BLK, _N), 1)
        ohm = (pr_ref[lo:hi, 0:1] == iota).astype(jnp.float32)
        ohn = (pr_ref[lo:hi, 1:2] == iota).astype(jnp.float32)
        lamn = 0.5 * jnp.dot(ohm + ohn, s, preferred_element_type=jnp.float32)
        lam = -jnp.sqrt(d2[lo:hi, :] + 1e-12) + alpha + lamn
        y = q1 * jnp.exp(lam) + q2 * lam
        out_ref[lo:hi, :] = jax.nn.sigmoid(y)


def kernel(node_pairs, adj_matrix, event_history, neighbor_data, node_embeds,
           W_proj, W_beta, b_beta, decay_theta, q1, q2):
    del adj_matrix
    f32 = jnp.float32
    i32 = jnp.int32
    E = node_embeds.astype(f32)
    ev = event_history.astype(f32)
    theta = jnp.reshape(decay_theta.astype(f32), (1, 1))
    q1r = jnp.reshape(jnp.asarray(q1, f32), (1, 1))
    q2r = jnp.reshape(jnp.asarray(q2, f32), (1, 1))

    half = jnp.arange(16, dtype=i32)
    intra = jnp.stack([half, half + 16], axis=1).reshape(32)
    perm = (jnp.arange(4, dtype=i32)[:, None] * 32 + intra[None, :]).reshape(
        _D)
    ebf16 = E[:, perm].astype(jnp.bfloat16)
    ebf = jax.lax.bitcast_convert_type(
        ebf16.reshape(_N, _D // 2, 2), jnp.int32)

    pairs = node_pairs.astype(i32)
    idxp = neighbor_data.astype(i32).reshape(-1)
    mnflat = pairs.reshape(-1)

    g, de = _sc_stage(ebf, idxp, mnflat)

    out = pl.pallas_call(
        _dense_stage,
        out_shape=jax.ShapeDtypeStruct((_P, 1), f32),
    )(ev, g, de, pairs, W_proj.astype(f32), W_beta.astype(f32),
      b_beta.astype(f32), theta, q1r, q2r)
    return out.reshape(_P)


# speedup vs baseline: 1.0082x; 1.0082x over previous
import functools

import jax
import jax.numpy as jnp
from jax import lax
from jax.experimental import pallas as pl
from jax.experimental.pallas import tpu as pltpu
from jax.experimental.pallas import tpu_sc as plsc

_N = 1024
_D = 128
_P = 4096
_R = 3
_K = 16
_RK = _R * _K
_CURRENT_TIME = 200.0

_NW = 32
_NODES_W = _N // _NW
_PAIRS_W = _P // _NW
_NSTEP = 8
_NPS = _NODES_W // _NSTEP
_ROWS = _NPS * _RK
_PAIR_BLK = 512


def _make_sc_kernel():
    mesh = plsc.VectorSubcoreMesh(core_axis_name="c", subcore_axis_name="s")

    @functools.partial(
        pl.kernel, mesh=mesh,
        compiler_params=pltpu.CompilerParams(
            needs_layout_passes=False, use_tc_tiling_on_sc=False),
        out_type=[jax.ShapeDtypeStruct((_R * _N, _D), jnp.float32),
                  jax.ShapeDtypeStruct((_P, _D), jnp.float32)],
        scratch_types=[
            pltpu.VMEM((_NODES_W * _RK,), jnp.int32),
            pltpu.VMEM((_ROWS, _D // 2), jnp.int32),
            pltpu.VMEM((_ROWS, _D // 2), jnp.int32),
            pltpu.VMEM((2 * _PAIRS_W,), jnp.int32),
            pltpu.VMEM((2 * _PAIRS_W, _D // 2), jnp.int32),
            pltpu.VMEM((_PAIRS_W, _D), jnp.float32),
            pltpu.VMEM((_R * _NODES_W, _D), jnp.float32),
            pltpu.SemaphoreType.DMA,
            pltpu.SemaphoreType.DMA,
            pltpu.SemaphoreType.DMA,
            pltpu.SemaphoreType.DMA,
        ],
    )
    def sc_body(ebf_hbm, idx_hbm, mn_hbm, g_hbm, de_hbm,
                idx_v, rows0, rows1, mn_v, prow_v, de_v, gbuf,
                sem0, sem1, semp0, semp1):
        wid = lax.axis_index("s") * 2 + lax.axis_index("c")
        pltpu.sync_copy(mn_hbm.at[pl.ds(wid * 2 * _PAIRS_W, 2 * _PAIRS_W)],
                        mn_v)
        cpp0 = pltpu.async_copy(ebf_hbm.at[mn_v.at[pl.ds(0, _PAIRS_W)]],
                                prow_v.at[pl.ds(0, _PAIRS_W)], semp0)
        cpp1 = pltpu.async_copy(
            ebf_hbm.at[mn_v.at[pl.ds(_PAIRS_W, _PAIRS_W)]],
            prow_v.at[pl.ds(_PAIRS_W, _PAIRS_W)], semp1)
        pltpu.sync_copy(idx_hbm.at[pl.ds(wid * _NODES_W * _RK,
                                         _NODES_W * _RK)], idx_v)

        bufs = (rows0, rows1)
        sems = (sem0, sem1)

        def fire(t):
            return pltpu.async_copy(
                ebf_hbm.at[idx_v.at[pl.ds(t * _ROWS, _ROWS)]],
                bufs[t % 2], sems[t % 2])

        cps = {0: fire(0), 1: fire(1)}
        for t in range(_NSTEP):
            cps[t].wait()
            buf = bufs[t % 2]

            def grp_body(g, _, _buf=buf, _t=t):
                u = g // _R
                r = g - u * _R
                base = g * _K

                def row_blk(j, accs, _buf=_buf, _base=base):
                    out = list(accs)
                    for jj in range(4):
                        row = _base + j * 4 + jj
                        for c in range(4):
                            w = _buf[row, pl.ds(c * 16, 16)]
                            v = plsc.bitcast(w, jnp.bfloat16)
                            a, b = plsc.unpack(
                                v, format=plsc.PackFormat.INTERLEAVED)
                            out[2 * c] = out[2 * c] + a
                            out[2 * c + 1] = out[2 * c + 1] + b
                    return tuple(out)

                accs = tuple(jnp.zeros((16,), jnp.float32) for _ in range(8))
                accs = lax.fori_loop(0, _K // 4, row_blk, accs)
                orow = r * _NODES_W + _t * _NPS + u
                for c in range(8):
                    gbuf[orow, pl.ds(c * 16, 16)] = accs[c]
                return 0

            lax.fori_loop(0, _NPS * _R, grp_body, 0)
            if t + 2 < _NSTEP:
                cps[t + 2] = fire(t + 2)

        for r in range(_R):
            pltpu.sync_copy(
                gbuf.at[pl.ds(r * _NODES_W, _NODES_W)],
                g_hbm.at[pl.ds(r * _N + wid * _NODES_W, _NODES_W)])

        cpp0.wait()
        cpp1.wait()

        def pair_body(p, _):
            for c in range(4):
                wa = prow_v[2 * p, pl.ds(c * 16, 16)]
                wb = prow_v[2 * p + 1, pl.ds(c * 16, 16)]
                a1, a2 = plsc.unpack(plsc.bitcast(wa, jnp.bfloat16),
                                     format=plsc.PackFormat.INTERLEAVED)
                b1, b2 = plsc.unpack(plsc.bitcast(wb, jnp.bfloat16),
                                     format=plsc.PackFormat.INTERLEAVED)
                de_v[p, pl.ds(c * 32, 16)] = a1 - b1
                de_v[p, pl.ds(c * 32 + 16, 16)] = a2 - b2
            return 0

        lax.fori_loop(0, _PAIRS_W, pair_body, 0)
        pltpu.sync_copy(de_v, de_hbm.at[pl.ds(wid * _PAIRS_W, _PAIRS_W)])

    return sc_body


_sc_stage = _make_sc_kernel()


def _dense_stage(ev_ref, g_ref, de_ref, pr_ref, Wp_ref, Wb_ref, bb_ref,
                 theta_ref, q1_ref, q2_ref, out_ref):
    theta = theta_ref[0, 0]
    alpha = jnp.sum(jnp.exp(-theta * (_CURRENT_TIME - ev_ref[...])))
    x = jnp.zeros((_N, _D), jnp.float32)
    for r in range(_R):
        x = x + jnp.dot(g_ref[r * _N:(r + 1) * _N, :], Wb_ref[r],
                        preferred_element_type=jnp.float32)
    bbar = jnp.mean(bb_ref[...], axis=0, keepdims=True)
    x = x * (1.0 / _RK) + bbar
    s = jnp.mean(jax.nn.sigmoid(x), axis=1, keepdims=True)
    diffs = jnp.dot(de_ref[...], Wp_ref[...],
                    preferred_element_type=jnp.float32)
    d2 = jnp.sum(diffs * diffs, axis=1, keepdims=True)
    q1 = q1_ref[0, 0]
    q2 = q2_ref[0, 0]
    for blk in range(_P // _PAIR_BLK):
        lo, hi = blk * _PAIR_BLK, (blk + 1) * _PAIR_BLK
        iota = lax.broadcasted_iota(jnp.int32, (_PAIR_BLK, _N), 1)
        ohm = (pr_ref[lo:hi, 0:1] == iota).astype(jnp.float32)
        ohn = (pr_ref[lo:hi, 1:2] == iota).astype(jnp.float32)
        lamn = 0.5 * jnp.dot(ohm + ohn, s, preferred_element_type=jnp.float32)
        lam = -jnp.sqrt(d2[lo:hi, :] + 1e-12) + alpha + lamn
        y = q1 * jnp.exp(lam) + q2 * lam
        out_ref[lo:hi, :] = jax.nn.sigmoid(y)


def kernel(node_pairs, adj_matrix, event_history, neighbor_data, node_embeds,
           W_proj, W_beta, b_beta, decay_theta, q1, q2):
    del adj_matrix
    f32 = jnp.float32
    i32 = jnp.int32
    E = node_embeds.astype(f32)
    ev = event_history.astype(f32)
    theta = jnp.reshape(decay_theta.astype(f32), (1, 1))
    q1r = jnp.reshape(jnp.asarray(q1, f32), (1, 1))
    q2r = jnp.reshape(jnp.asarray(q2, f32), (1, 1))

    half = jnp.arange(16, dtype=i32)
    intra = jnp.stack([half, half + 16], axis=1).reshape(32)
    perm = (jnp.arange(4, dtype=i32)[:, None] * 32 + intra[None, :]).reshape(
        _D)
    ebf16 = E[:, perm].astype(jnp.bfloat16)
    ebf = jax.lax.bitcast_convert_type(
        ebf16.reshape(_N, _D // 2, 2), jnp.int32)

    pairs = node_pairs.astype(i32)
    idxp = neighbor_data.astype(i32).reshape(-1)
    mnflat = pairs.reshape(-1)

    g, de = _sc_stage(ebf, idxp, mnflat)

    out = pl.pallas_call(
        _dense_stage,
        out_shape=jax.ShapeDtypeStruct((_P, 1), f32),
    )(ev, g, de, pairs, W_proj.astype(f32), W_beta.astype(f32),
      b_beta.astype(f32), theta, q1r, q2r)
    return out.reshape(_P)
